# baseline (speedup 1.0000x reference)
import jax
import jax.numpy as jnp
from jax.experimental import pallas as pl

BATCH = 10000
DIM = 512
ROWS_PER_BLOCK = 10000


def _copy_body(x_ref, o_ref):
    o_ref[...] = x_ref[...]


def kernel(x, ind, mask, sampled, embed):
    del ind, mask, sampled, embed
    return pl.pallas_call(
        _copy_body,
        grid=(BATCH // ROWS_PER_BLOCK,),
        in_specs=[pl.BlockSpec((ROWS_PER_BLOCK, DIM), lambda i: (i, 0))],
        out_specs=pl.BlockSpec((ROWS_PER_BLOCK, DIM), lambda i: (i, 0)),
        out_shape=jax.ShapeDtypeStruct((BATCH, DIM), jnp.float32),
    )(x)


# speedup vs baseline: 1.0651x; 1.0651x over previous
import jax
import jax.numpy as jnp
from jax.experimental import pallas as pl
from jax.experimental.pallas import tpu as pltpu

BATCH = 10000
DIM = 512
CHUNK = 2000
NCHUNK = BATCH // CHUNK
NBUF = 3


def _stream_body(x_hbm, o_hbm, buf, sem_in, sem_out):
    def in_copy(i, slot):
        return pltpu.make_async_copy(
            x_hbm.at[pl.ds(i * CHUNK, CHUNK), :], buf.at[slot], sem_in.at[slot]
        )

    def out_copy(i, slot):
        return pltpu.make_async_copy(
            buf.at[slot], o_hbm.at[pl.ds(i * CHUNK, CHUNK), :], sem_out.at[slot]
        )

    for i in range(min(NBUF, NCHUNK)):
        in_copy(i, i).start()
    for i in range(NCHUNK):
        slot = i % NBUF
        in_copy(i, slot).wait()
        out_copy(i, slot).start()
        nxt = i + NBUF
        if nxt < NCHUNK:
            out_copy(i, slot).wait()
            in_copy(nxt, slot).start()
    for i in range(max(NCHUNK - NBUF, 0), NCHUNK):
        out_copy(i, i % NBUF).wait()


def kernel(x, ind, mask, sampled, embed):
    del ind, mask, sampled, embed
    return pl.pallas_call(
        _stream_body,
        in_specs=[pl.BlockSpec(memory_space=pltpu.MemorySpace.HBM)],
        out_specs=pl.BlockSpec(memory_space=pltpu.MemorySpace.HBM),
        out_shape=jax.ShapeDtypeStruct((BATCH, DIM), jnp.float32),
        scratch_shapes=[
            pltpu.VMEM((NBUF, CHUNK, DIM), jnp.float32),
            pltpu.SemaphoreType.DMA((NBUF,)),
            pltpu.SemaphoreType.DMA((NBUF,)),
        ],
    )(x)


# speedup vs baseline: 1.1513x; 1.0810x over previous
import jax
import jax.numpy as jnp
from jax.experimental import pallas as pl
from jax.experimental.pallas import tpu as pltpu

BATCH = 10000
DIM = 512
CHUNK = 5000
NCHUNK = BATCH // CHUNK
NBUF = 2


def _stream_body(x_hbm, o_hbm, buf, sem_in, sem_out):
    def in_copy(i, slot):
        return pltpu.make_async_copy(
            x_hbm.at[pl.ds(i * CHUNK, CHUNK), :], buf.at[slot], sem_in.at[slot]
        )

    def out_copy(i, slot):
        return pltpu.make_async_copy(
            buf.at[slot], o_hbm.at[pl.ds(i * CHUNK, CHUNK), :], sem_out.at[slot]
        )

    for i in range(min(NBUF, NCHUNK)):
        in_copy(i, i).start()
    for i in range(NCHUNK):
        slot = i % NBUF
        in_copy(i, slot).wait()
        out_copy(i, slot).start()
        nxt = i + NBUF
        if nxt < NCHUNK:
            out_copy(i, slot).wait()
            in_copy(nxt, slot).start()
    for i in range(max(NCHUNK - NBUF, 0), NCHUNK):
        out_copy(i, i % NBUF).wait()


def kernel(x, ind, mask, sampled, embed):
    del ind, mask, sampled, embed
    return pl.pallas_call(
        _stream_body,
        in_specs=[pl.BlockSpec(memory_space=pltpu.MemorySpace.HBM)],
        out_specs=pl.BlockSpec(memory_space=pltpu.MemorySpace.HBM),
        out_shape=jax.ShapeDtypeStruct((BATCH, DIM), jnp.float32),
        scratch_shapes=[
            pltpu.VMEM((NBUF, CHUNK, DIM), jnp.float32),
            pltpu.SemaphoreType.DMA((NBUF,)),
            pltpu.SemaphoreType.DMA((NBUF,)),
        ],
    )(x)
